# baseline (speedup 1.0000x reference)
import functools

import jax
import jax.numpy as jnp
from jax import lax
from jax.experimental import pallas as pl
from jax.experimental.pallas import tpu as pltpu
from jax.experimental.pallas import tpu_sc as plsc

_NC = 2
_NS = 16
_NW = _NC * _NS


def _sc_gather_add(x2d, idx, table):
    V, D = table.shape
    N = idx.shape[0]
    n_per_w = N // _NW
    R = 16
    n_chunks = n_per_w // R
    n_main = n_chunks - 2
    mesh = plsc.VectorSubcoreMesh(core_axis_name="c", subcore_axis_name="s")

    @functools.partial(
        pl.kernel, mesh=mesh,
        out_type=jax.ShapeDtypeStruct((N, D), jnp.float32),
        scratch_types=[
            pltpu.VMEM((n_per_w,), jnp.int32),
            pltpu.VMEM((3, R, D), jnp.float32),
            pltpu.VMEM((3, R, D), jnp.float32),
            pltpu.SemaphoreType.DMA((3,)),
            pltpu.SemaphoreType.DMA((3,)),
            pltpu.SemaphoreType.DMA((3,)),
        ],
    )
    def k(x_hbm, idx_hbm, table_hbm, out_hbm, idx_v, o_v, pe_v,
          xsem, gsem, osem):
        wid = lax.axis_index("s") * _NC + lax.axis_index("c")
        base = wid * n_per_w
        pltpu.sync_copy(idx_hbm.at[pl.ds(base, n_per_w)], idx_v)

        def start_fetch(c, b):
            pltpu.async_copy(
                x_hbm.at[pl.ds(base + c * R, R)], o_v.at[b], xsem.at[b])
            pltpu.async_copy(
                table_hbm.at[idx_v.at[pl.ds(c * R, R)]], pe_v.at[b],
                gsem.at[b])

        def wait_fetch(b):
            pltpu.make_async_copy(
                x_hbm.at[pl.ds(0, R)], o_v.at[b], xsem.at[b]).wait()
            pltpu.make_async_copy(
                table_hbm.at[pl.ds(0, R)], pe_v.at[b], gsem.at[b]).wait()

        def wait_out(b):
            pltpu.make_async_copy(
                o_v.at[b], out_hbm.at[pl.ds(0, R)], osem.at[b]).wait()

        def body(cc, b):
            wait_fetch(b)

            @pl.loop(0, R)
            def _(r):
                @pl.loop(0, D, step=64)
                def _(col):
                    for u in range(4):
                        s = pl.ds(col + u * 16, 16)
                        plsc.addupdate(o_v.at[b, r, s], pe_v.at[b, r, s][...])

            pltpu.async_copy(
                o_v.at[b], out_hbm.at[pl.ds(base + cc * R, R)], osem.at[b])

        start_fetch(0, 0)
        start_fetch(1, 1)

        @pl.loop(0, n_main, step=3)
        def _(c):
            for b0 in range(3):
                cc = c + b0
                body(cc, b0)
                b2 = (b0 + 2) % 3

                @pl.when(cc + 2 < n_chunks)
                def _():
                    @pl.when(cc >= 1)
                    def _():
                        wait_out(b2)
                    start_fetch(cc + 2, b2)

        body(n_chunks - 2, (n_chunks - 2) % 3)
        body(n_chunks - 1, (n_chunks - 1) % 3)
        wait_out((n_chunks - 3) % 3)
        wait_out((n_chunks - 2) % 3)
        wait_out((n_chunks - 1) % 3)

    return k(x2d, idx, table)


def kernel(x, position_indices, pe_table):
    B, S, D = x.shape
    idx = position_indices.reshape(-1).astype(jnp.int32)
    out2d = _sc_gather_add(x.reshape(B * S, D), idx, pe_table)
    return out2d.reshape(B, S, D)


# speedup vs baseline: 1.0031x; 1.0031x over previous
import functools

import jax
import jax.numpy as jnp
from jax import lax
from jax.experimental import pallas as pl
from jax.experimental.pallas import tpu as pltpu
from jax.experimental.pallas import tpu_sc as plsc

_NC = 2
_NS = 16
_NW = _NC * _NS


def _sc_gather_add(x, idx, table):
    B, S, D = x.shape
    V, _ = table.shape
    n_per_w = (B * S) // _NW
    R = 16
    n_chunks = n_per_w // R
    mesh = plsc.VectorSubcoreMesh(core_axis_name="c", subcore_axis_name="s")

    @functools.partial(
        pl.kernel, mesh=mesh,
        out_type=jax.ShapeDtypeStruct((B, S, D), jnp.float32),
        scratch_types=[
            pltpu.VMEM((n_per_w,), jnp.int32),
            pltpu.VMEM((2, R, D), jnp.float32),
            pltpu.VMEM((2, R, D), jnp.float32),
            pltpu.VMEM((2, R, D), jnp.float32),
            pltpu.SemaphoreType.DMA,
            pltpu.SemaphoreType.DMA,
            pltpu.SemaphoreType.DMA,
            pltpu.SemaphoreType.DMA,
            pltpu.SemaphoreType.DMA,
            pltpu.SemaphoreType.DMA,
        ],
    )
    def k(x_hbm, idx_hbm, table_hbm, out_hbm, idx_v, pe_v, x_v, o_v,
          gs0, gs1, xs0, xs1, os0, os1):
        gsem = (gs0, gs1)
        xsem = (xs0, xs1)
        osem = (os0, os1)
        wid = lax.axis_index("s") * _NC + lax.axis_index("c")
        bi = wid // (S // n_per_w)
        soff = (wid % (S // n_per_w)) * n_per_w
        pltpu.sync_copy(idx_hbm.at[bi, pl.ds(soff, n_per_w)], idx_v)

        def start_fetch(c, b):
            pltpu.async_copy(
                table_hbm.at[idx_v.at[pl.ds(c * R, R)]], pe_v.at[b], gsem[b])
            pltpu.async_copy(
                x_hbm.at[bi, pl.ds(soff + c * R, R)], x_v.at[b], xsem[b])

        def wait_fetch(b):
            pltpu.make_async_copy(
                table_hbm.at[pl.ds(0, R)], pe_v.at[b], gsem[b]).wait()
            pltpu.make_async_copy(
                x_hbm.at[0, pl.ds(0, R)], x_v.at[b], xsem[b]).wait()

        def wait_out(b):
            pltpu.make_async_copy(
                o_v.at[b], out_hbm.at[0, pl.ds(0, R)], osem[b]).wait()

        start_fetch(0, 0)
        start_fetch(1, 1)

        @pl.loop(0, n_chunks, step=2)
        def _(c):
            for b in range(2):
                cc = c + b
                wait_fetch(b)

                @pl.when(cc >= 2)
                def _():
                    wait_out(b)

                @pl.loop(0, R)
                def _(r):
                    @pl.loop(0, D, step=64)
                    def _(col):
                        for u in range(4):
                            s = pl.ds(col + u * 16, 16)
                            o_v.at[b, r, s][...] = (
                                pe_v.at[b, r, s][...] + x_v.at[b, r, s][...])

                pltpu.async_copy(
                    o_v.at[b], out_hbm.at[bi, pl.ds(soff + cc * R, R)],
                    osem[b])

                @pl.when(cc + 2 < n_chunks)
                def _():
                    start_fetch(cc + 2, b)

        wait_out(0)
        wait_out(1)

    return k(x, idx, table)


def kernel(x, position_indices, pe_table):
    return _sc_gather_add(x, position_indices.astype(jnp.int32), pe_table)


# speedup vs baseline: 1.0075x; 1.0044x over previous
import functools

import jax
import jax.numpy as jnp
from jax import lax
from jax.experimental import pallas as pl
from jax.experimental.pallas import tpu as pltpu
from jax.experimental.pallas import tpu_sc as plsc

_NC = 2
_NS = 16
_NW = _NC * _NS


def _sc_gather_add(x, idx, table):
    B, S, D = x.shape
    V, _ = table.shape
    n_per_w = (B * S) // _NW
    R = 16
    n_chunks = n_per_w // R
    mesh = plsc.VectorSubcoreMesh(core_axis_name="c", subcore_axis_name="s")

    @functools.partial(
        pl.kernel, mesh=mesh,
        out_type=jax.ShapeDtypeStruct((B, S, D), jnp.float32),
        scratch_types=[
            pltpu.VMEM((n_per_w,), jnp.int32),
            pltpu.VMEM((2, R, D), jnp.float32),
            pltpu.VMEM((2, R, D), jnp.float32),
            pltpu.VMEM((2, R, D), jnp.float32),
            pltpu.SemaphoreType.DMA,
            pltpu.SemaphoreType.DMA,
            pltpu.SemaphoreType.DMA,
            pltpu.SemaphoreType.DMA,
            pltpu.SemaphoreType.DMA,
            pltpu.SemaphoreType.DMA,
        ],
    )
    def k(x_hbm, idx_hbm, table_hbm, out_hbm, idx_v, pe_v, x_v, o_v,
          gs0, gs1, xs0, xs1, os0, os1):
        gsem = (gs0, gs1)
        xsem = (xs0, xs1)
        osem = (os0, os1)
        wid = lax.axis_index("s") * _NC + lax.axis_index("c")
        bi = wid // (S // n_per_w)
        soff = (wid % (S // n_per_w)) * n_per_w
        def start_fetch(c, b):
            pltpu.async_copy(
                table_hbm.at[idx_v.at[pl.ds(c * R, R)]], pe_v.at[b], gsem[b])
            pltpu.async_copy(
                x_hbm.at[bi, pl.ds(soff + c * R, R)], x_v.at[b], xsem[b])

        def start_x(c, b):
            pltpu.async_copy(
                x_hbm.at[bi, pl.ds(soff + c * R, R)], x_v.at[b], xsem[b])

        def start_gather(c, b):
            pltpu.async_copy(
                table_hbm.at[idx_v.at[pl.ds(c * R, R)]], pe_v.at[b], gsem[b])

        def wait_fetch(b):
            pltpu.make_async_copy(
                table_hbm.at[pl.ds(0, R)], pe_v.at[b], gsem[b]).wait()
            pltpu.make_async_copy(
                x_hbm.at[0, pl.ds(0, R)], x_v.at[b], xsem[b]).wait()

        def wait_out(b):
            pltpu.make_async_copy(
                o_v.at[b], out_hbm.at[0, pl.ds(0, R)], osem[b]).wait()

        start_x(0, 0)
        start_x(1, 1)
        pltpu.sync_copy(idx_hbm.at[bi, pl.ds(soff, n_per_w)], idx_v)
        start_gather(0, 0)
        start_gather(1, 1)

        @pl.loop(0, n_chunks, step=2)
        def _(c):
            for b in range(2):
                cc = c + b
                wait_fetch(b)

                @pl.when(cc >= 2)
                def _():
                    wait_out(b)

                @pl.loop(0, R)
                def _(r):
                    @pl.loop(0, D, step=64)
                    def _(col):
                        for u in range(4):
                            s = pl.ds(col + u * 16, 16)
                            o_v.at[b, r, s][...] = (
                                pe_v.at[b, r, s][...] + x_v.at[b, r, s][...])

                pltpu.async_copy(
                    o_v.at[b], out_hbm.at[bi, pl.ds(soff + cc * R, R)],
                    osem[b])

                @pl.when(cc + 2 < n_chunks)
                def _():
                    start_fetch(cc + 2, b)

        wait_out(0)
        wait_out(1)

    return k(x, idx, table)


def kernel(x, position_indices, pe_table):
    return _sc_gather_add(x, position_indices.astype(jnp.int32), pe_table)
